# baseline (speedup 1.0000x reference)
import jax
import jax.numpy as jnp
from jax.experimental import pallas as pl
from jax.experimental.pallas import tpu as pltpu

_C = 32
_T = 4
_F = 128
_L = 512
_N = _F * _L
_K = 128
_D = _C * _T
_INT_MIN = -2147483648


def _fold(mask3):
    v = jnp.where(mask3, 1.0, 0.0)
    acc = v[0]
    for g in range(1, 16):
        acc = acc + v[g]
    for sh in (4, 2, 1):
        acc = acc + pltpu.roll(acc, sh, axis=0)
    return acc


def _tile4(p):
    return jnp.concatenate([p, p, p, p], axis=1)


def _counts(masks, ones_bf):
    accs = [_fold(m).astype(jnp.bfloat16) for m in masks]
    stacked = jnp.concatenate(accs, axis=0)
    r = jax.lax.dot_general(stacked, ones_bf, (((1,), (0,)), ((), ())),
                            preferred_element_type=jnp.float32)
    return [_tile4(r[8 * i:8 * (i + 1)]) for i in range(len(masks))]


def _count(mask3, ones_bf):
    return _counts([mask3], ones_bf)[0]


def _attn_kernel(x_ref, w_ref, out_ref):
    Xm = x_ref[0]
    w = w_ref[0:1, 0:1]

    blocks = [Xm[:, f * _L:(f + 1) * _L] for f in range(_F)]
    while len(blocks) > 1:
        blocks = [blocks[i] + blocks[i + 1] for i in range(0, len(blocks), 2)]
    colsum = blocks[0]
    total = jnp.sum(colsum, axis=1, keepdims=True)
    last = colsum[:, _L - 1:_L]
    qs = []
    prefix = jnp.zeros_like(total)
    for i in range(_T):
        qs.append((total - prefix + i * last) * (1.0 / _N))
        if i < _T - 1:
            prefix = prefix + colsum[:, i:i + 1]
    Qt = jnp.concatenate(qs, axis=1)

    s = jax.lax.dot_general(Qt, Xm, (((0,), (0,)), ((), ())),
                            preferred_element_type=jnp.float32)
    s3 = s.reshape(_T, _F, _L)

    score = s3[0]
    for i in range(1, _T):
        body = s3[i, :, i:]
        tail = jnp.broadcast_to(s3[i, :, _L - 1:_L], (_F, i))
        score = score + jnp.concatenate([body, tail], axis=1)

    bits = jax.lax.bitcast_convert_type(score, jnp.int32)
    key = jnp.where(bits >= 0, bits, jnp.int32(_INT_MIN) - bits)
    key3 = key.reshape(16, 8, _L)

    ones_bf = jnp.ones((_L, 128), jnp.bfloat16)
    ones_f32 = jnp.ones((_L, 128), jnp.float32)

    t = jnp.where(_count(key3 >= 0, ones_bf) >= _K, 0,
                  _INT_MIN).astype(jnp.int32)
    bit_pairs = [(b, b - 1) for b in range(30, 0, -2)] + [(0, None)]
    for b_hi, b_lo in bit_pairs:
        if b_lo is None:
            cand = t | (1 << b_hi)
            t = jnp.where(_count(key3 >= cand[None], ones_bf) >= _K, cand, t)
        else:
            u1 = t | (1 << b_lo)
            u2 = t | (1 << b_hi)
            u3 = u2 | (1 << b_lo)
            n1, n2, n3 = _counts([key3 >= u1[None], key3 >= u2[None],
                                  key3 >= u3[None]], ones_bf)
            t = jnp.where(n1 >= _K, u1, t)
            t = jnp.where(n2 >= _K, u2, t)
            t = jnp.where(n3 >= _K, u3, t)

    gt3 = key3 > t[None]
    eq3 = key3 == t[None]
    need = _K - _count(gt3, ones_bf)
    idx3 = (jax.lax.broadcasted_iota(jnp.int32, (16, 8, _L), 0) * (8 * _L)
            + jax.lax.broadcasted_iota(jnp.int32, (16, 8, _L), 1) * _L
            + jax.lax.broadcasted_iota(jnp.int32, (16, 8, _L), 2))

    jm = jnp.zeros((8, _L), jnp.int32)
    j_pairs = [(b, b - 1) for b in range(16, 0, -2)] + [(0, None)]
    for b_hi, b_lo in j_pairs:
        if b_lo is None:
            cand = jm | (1 << b_hi)
            c = _count(eq3 & (idx3 < cand[None]), ones_bf)
            jm = jnp.where(c < need, cand, jm)
        else:
            v1 = jm | (1 << b_lo)
            v2 = jm | (1 << b_hi)
            v3 = v2 | (1 << b_lo)
            c1, c2, c3 = _counts([eq3 & (idx3 < v1[None]),
                                  eq3 & (idx3 < v2[None]),
                                  eq3 & (idx3 < v3[None])], ones_bf)
            jm = jnp.where(c1 < need, v1, jm)
            jm = jnp.where(c2 < need, v2, jm)
            jm = jnp.where(c3 < need, v3, jm)
    mask3 = gt3 | (eq3 & (idx3 <= jm[None]))

    mbits = jnp.where(t > 0, t, jnp.int32(_INT_MIN) - t)
    mshift = jax.lax.bitcast_convert_type(mbits, jnp.float32)

    score3 = score.reshape(16, 8, _L)
    e3 = jnp.where(mask3,
                   jnp.exp(jnp.minimum(score3 - mshift[None], 80.0)), 0.0)
    zacc = e3[0]
    for g in range(1, 16):
        zacc = zacc + e3[g]
    for sh in (4, 2, 1):
        zacc = zacc + pltpu.roll(zacc, sh, axis=0)
    Z = jax.lax.dot_general(zacc, ones_f32, (((1,), (0,)), ((), ())),
                            preferred_element_type=jnp.float32)
    e = e3.reshape(_F, _L)

    As = []
    for i in range(_T):
        if i == 0:
            As.append(e)
        else:
            zeros = jnp.zeros((_F, i), jnp.float32)
            bodyp = e[:, :_L - 1 - i]
            tailp = jnp.sum(e[:, _L - 1 - i:], axis=1, keepdims=True)
            As.append(jnp.concatenate([zeros, bodyp, tailp], axis=1))
    A = jnp.stack(As, axis=0).reshape(_T, _N)

    outdot = jax.lax.dot_general(Xm, A, (((1,), (1,)), ((), ())),
                                 preferred_element_type=jnp.float32)
    G = (w / Z[0:1, 0:1]) * outdot + (0.5 - w) * Qt
    out_ref[0] = G.T


def kernel(x, w):
    B = x.shape[0]
    xm = x.reshape(B, _C, _N)
    w2 = jnp.asarray(w, jnp.float32).reshape(1, 1)
    out = pl.pallas_call(
        _attn_kernel,
        grid=(B,),
        in_specs=[
            pl.BlockSpec((1, _C, _N), lambda b: (b, 0, 0)),
            pl.BlockSpec((1, 1), lambda b: (0, 0)),
        ],
        out_specs=pl.BlockSpec((1, _T, _C), lambda b: (b, 0, 0)),
        out_shape=jax.ShapeDtypeStruct((B, _T, _C), jnp.float32),
        compiler_params=pltpu.CompilerParams(
            dimension_semantics=("parallel",)),
    )(xm, w2)
    return out.reshape(B, _C, 1, _T)


# speedup vs baseline: 1.4803x; 1.4803x over previous
import jax
import jax.numpy as jnp
from jax.experimental import pallas as pl
from jax.experimental.pallas import tpu as pltpu

_C = 32
_T = 4
_F = 128
_L = 512
_N = _F * _L
_K = 128
_D = _C * _T
_INT_MIN = -2147483648


def _fold(mask3):
    v = jnp.where(mask3, 1.0, 0.0)
    acc = v[0]
    for g in range(1, 16):
        acc = acc + v[g]
    for sh in (4, 2, 1):
        acc = acc + pltpu.roll(acc, sh, axis=0)
    return acc


def _tile4(p):
    return jnp.concatenate([p, p, p, p], axis=1)


def _counts(masks, ones_bf):
    accs = [_fold(m).astype(jnp.bfloat16) for m in masks]
    stacked = jnp.concatenate(accs, axis=0)
    r = jax.lax.dot_general(stacked, ones_bf, (((1,), (0,)), ((), ())),
                            preferred_element_type=jnp.float32)
    return [_tile4(r[8 * i:8 * (i + 1)]) for i in range(len(masks))]


def _count(mask3, ones_bf):
    return _counts([mask3], ones_bf)[0]


def _attn_kernel(x_ref, w_ref, out_ref):
    X = x_ref[0]
    w = w_ref[0:1, 0:1]

    colsum = jnp.sum(X, axis=1)
    total = jnp.sum(colsum, axis=1, keepdims=True)
    last = colsum[:, _L - 1:_L]
    qs = []
    prefix = jnp.zeros_like(total)
    for i in range(_T):
        qs.append((total - prefix + i * last) * (1.0 / _N))
        if i < _T - 1:
            prefix = prefix + colsum[:, i:i + 1]
    Qt = jnp.concatenate(qs, axis=1)

    Xm = X.reshape(_C, _N)
    s = jax.lax.dot_general(Qt, Xm, (((0,), (0,)), ((), ())),
                            preferred_element_type=jnp.float32)
    s3 = s.reshape(_T, _F, _L)

    score = s3[0]
    for i in range(1, _T):
        body = s3[i, :, i:]
        tail = jnp.broadcast_to(s3[i, :, _L - 1:_L], (_F, i))
        score = score + jnp.concatenate([body, tail], axis=1)

    bits = jax.lax.bitcast_convert_type(score, jnp.int32)
    key = jnp.where(bits >= 0, bits, jnp.int32(_INT_MIN) - bits)
    key3 = key.reshape(16, 8, _L)

    ones_bf = jnp.ones((_L, 128), jnp.bfloat16)
    ones_f32 = jnp.ones((_L, 128), jnp.float32)

    t = jnp.where(_count(key3 >= 0, ones_bf) >= _K, 0,
                  _INT_MIN).astype(jnp.int32)
    bit_pairs = [(b, b - 1) for b in range(30, 0, -2)] + [(0, None)]
    for b_hi, b_lo in bit_pairs:
        if b_lo is None:
            cand = t | (1 << b_hi)
            t = jnp.where(_count(key3 >= cand[None], ones_bf) >= _K, cand, t)
        else:
            u1 = t | (1 << b_lo)
            u2 = t | (1 << b_hi)
            u3 = u2 | (1 << b_lo)
            n1, n2, n3 = _counts([key3 >= u1[None], key3 >= u2[None],
                                  key3 >= u3[None]], ones_bf)
            t = jnp.where(n1 >= _K, u1, t)
            t = jnp.where(n2 >= _K, u2, t)
            t = jnp.where(n3 >= _K, u3, t)

    gt3 = key3 > t[None]
    eq3 = key3 == t[None]
    need = _K - _count(gt3, ones_bf)
    idx3 = (jax.lax.broadcasted_iota(jnp.int32, (16, 8, _L), 0) * (8 * _L)
            + jax.lax.broadcasted_iota(jnp.int32, (16, 8, _L), 1) * _L
            + jax.lax.broadcasted_iota(jnp.int32, (16, 8, _L), 2))

    jm = jnp.zeros((8, _L), jnp.int32)
    j_pairs = [(b, b - 1) for b in range(16, 0, -2)] + [(0, None)]
    for b_hi, b_lo in j_pairs:
        if b_lo is None:
            cand = jm | (1 << b_hi)
            c = _count(eq3 & (idx3 < cand[None]), ones_bf)
            jm = jnp.where(c < need, cand, jm)
        else:
            v1 = jm | (1 << b_lo)
            v2 = jm | (1 << b_hi)
            v3 = v2 | (1 << b_lo)
            c1, c2, c3 = _counts([eq3 & (idx3 < v1[None]),
                                  eq3 & (idx3 < v2[None]),
                                  eq3 & (idx3 < v3[None])], ones_bf)
            jm = jnp.where(c1 < need, v1, jm)
            jm = jnp.where(c2 < need, v2, jm)
            jm = jnp.where(c3 < need, v3, jm)
    mask3 = gt3 | (eq3 & (idx3 <= jm[None]))

    mbits = jnp.where(t > 0, t, jnp.int32(_INT_MIN) - t)
    mshift = jax.lax.bitcast_convert_type(mbits, jnp.float32)

    score3 = score.reshape(16, 8, _L)
    e3 = jnp.where(mask3,
                   jnp.exp(jnp.minimum(score3 - mshift[None], 80.0)), 0.0)
    zacc = e3[0]
    for g in range(1, 16):
        zacc = zacc + e3[g]
    for sh in (4, 2, 1):
        zacc = zacc + pltpu.roll(zacc, sh, axis=0)
    Z = jax.lax.dot_general(zacc, ones_f32, (((1,), (0,)), ((), ())),
                            preferred_element_type=jnp.float32)
    e = e3.reshape(_F, _L)

    As = []
    for i in range(_T):
        if i == 0:
            As.append(e)
        else:
            zeros = jnp.zeros((_F, i), jnp.float32)
            bodyp = e[:, :_L - 1 - i]
            tailp = jnp.sum(e[:, _L - 1 - i:], axis=1, keepdims=True)
            As.append(jnp.concatenate([zeros, bodyp, tailp], axis=1))
    A = jnp.stack(As, axis=0).reshape(_T, _N)

    outdot = jax.lax.dot_general(Xm, A, (((1,), (1,)), ((), ())),
                                 preferred_element_type=jnp.float32)
    G = (w / Z[0:1, 0:1]) * outdot + (0.5 - w) * Qt
    out_ref[0] = G.T


def kernel(x, w):
    B = x.shape[0]
    w2 = jnp.asarray(w, jnp.float32).reshape(1, 1)
    out = pl.pallas_call(
        _attn_kernel,
        grid=(B,),
        in_specs=[
            pl.BlockSpec((1, _C, _F, _L), lambda b: (b, 0, 0, 0)),
            pl.BlockSpec((1, 1), lambda b: (0, 0)),
        ],
        out_specs=pl.BlockSpec((1, _T, _C), lambda b: (b, 0, 0)),
        out_shape=jax.ShapeDtypeStruct((B, _T, _C), jnp.float32),
        compiler_params=pltpu.CompilerParams(
            dimension_semantics=("parallel",)),
    )(x, w2)
    return out.reshape(B, _C, 1, _T)


# speedup vs baseline: 1.8307x; 1.2368x over previous
import jax
import jax.numpy as jnp
from jax.experimental import pallas as pl
from jax.experimental.pallas import tpu as pltpu

_C = 32
_T = 4
_F = 128
_L = 512
_N = _F * _L
_K = 128
_D = _C * _T
_INT_MIN = -2147483648


def _fold(mask3):
    v = jnp.where(mask3, 1.0, 0.0)
    acc = v[0]
    for g in range(1, 16):
        acc = acc + v[g]
    for sh in (4, 2, 1):
        acc = acc + pltpu.roll(acc, sh, axis=0)
    return acc


def _tile4(p):
    return jnp.concatenate([p, p, p, p], axis=1)


def _counts(masks, ones_bf):
    accs = [_fold(m).astype(jnp.bfloat16) for m in masks]
    stacked = jnp.concatenate(accs, axis=0)
    r = jax.lax.dot_general(stacked, ones_bf, (((1,), (0,)), ((), ())),
                            preferred_element_type=jnp.float32)
    return [_tile4(r[8 * i:8 * (i + 1)]) for i in range(len(masks))]


def _count(mask3, ones_bf):
    return _counts([mask3], ones_bf)[0]


def _attn_kernel(x_ref, w_ref, out_ref):
    X = x_ref[0]
    w = w_ref[0:1, 0:1]

    colsum = jnp.sum(X, axis=1)
    total = jnp.sum(colsum, axis=1, keepdims=True)
    last = colsum[:, _L - 1:_L]
    qs = []
    prefix = jnp.zeros_like(total)
    for i in range(_T):
        qs.append((total - prefix + i * last) * (1.0 / _N))
        if i < _T - 1:
            prefix = prefix + colsum[:, i:i + 1]
    Qt = jnp.concatenate(qs, axis=1)

    Xm = X.reshape(_C, _N)
    s = jax.lax.dot_general(Qt, Xm, (((0,), (0,)), ((), ())),
                            preferred_element_type=jnp.float32)
    s3 = s.reshape(_T, _F, _L)

    score = s3[0]
    for i in range(1, _T):
        body = s3[i, :, i:]
        tail = jnp.broadcast_to(s3[i, :, _L - 1:_L], (_F, i))
        score = score + jnp.concatenate([body, tail], axis=1)

    bits = jax.lax.bitcast_convert_type(score, jnp.int32)
    key = jnp.where(bits >= 0, bits, jnp.int32(_INT_MIN) - bits)
    key3 = key.reshape(16, 8, _L)

    ones_bf = jnp.ones((_L, 128), jnp.bfloat16)
    ones_f32 = jnp.ones((_L, 128), jnp.float32)

    t = jnp.where(_count(key3 >= 0, ones_bf) >= _K, 0,
                  _INT_MIN).astype(jnp.int32)
    bit_pairs = [(b, b - 1) for b in range(30, 0, -2)] + [(0, None)]
    for b_hi, b_lo in bit_pairs:
        if b_lo is None:
            cand = t | (1 << b_hi)
            t = jnp.where(_count(key3 >= cand[None], ones_bf) >= _K, cand, t)
        else:
            u1 = t | (1 << b_lo)
            u2 = t | (1 << b_hi)
            u3 = u2 | (1 << b_lo)
            n1, n2, n3 = _counts([key3 >= u1[None], key3 >= u2[None],
                                  key3 >= u3[None]], ones_bf)
            t = jnp.where(n1 >= _K, u1, t)
            t = jnp.where(n2 >= _K, u2, t)
            t = jnp.where(n3 >= _K, u3, t)

    gt3 = key3 > t[None]
    eq3 = key3 == t[None]
    need = _K - _count(gt3, ones_bf)
    idx3 = (jax.lax.broadcasted_iota(jnp.int32, (16, 8, _L), 0) * (8 * _L)
            + jax.lax.broadcasted_iota(jnp.int32, (16, 8, _L), 1) * _L
            + jax.lax.broadcasted_iota(jnp.int32, (16, 8, _L), 2))

    cnt_eq = _count(eq3, ones_bf)

    def _tie_search():
        jm = jnp.zeros((8, _L), jnp.int32)
        j_pairs = [(b, b - 1) for b in range(16, 0, -2)] + [(0, None)]
        for b_hi, b_lo in j_pairs:
            if b_lo is None:
                cand = jm | (1 << b_hi)
                c = _count(eq3 & (idx3 < cand[None]), ones_bf)
                jm = jnp.where(c < need, cand, jm)
            else:
                v1 = jm | (1 << b_lo)
                v2 = jm | (1 << b_hi)
                v3 = v2 | (1 << b_lo)
                c1, c2, c3 = _counts([eq3 & (idx3 < v1[None]),
                                      eq3 & (idx3 < v2[None]),
                                      eq3 & (idx3 < v3[None])], ones_bf)
                jm = jnp.where(c1 < need, v1, jm)
                jm = jnp.where(c2 < need, v2, jm)
                jm = jnp.where(c3 < need, v3, jm)
        return jm

    jm = jax.lax.cond(cnt_eq[0, 0] == need[0, 0],
                      lambda: jnp.full((8, _L), _N, jnp.int32),
                      _tie_search)
    mask3 = gt3 | (eq3 & (idx3 <= jm[None]))

    mbits = jnp.where(t > 0, t, jnp.int32(_INT_MIN) - t)
    mshift = jax.lax.bitcast_convert_type(mbits, jnp.float32)

    score3 = score.reshape(16, 8, _L)
    e3 = jnp.where(mask3,
                   jnp.exp(jnp.minimum(score3 - mshift[None], 80.0)), 0.0)
    zacc = e3[0]
    for g in range(1, 16):
        zacc = zacc + e3[g]
    for sh in (4, 2, 1):
        zacc = zacc + pltpu.roll(zacc, sh, axis=0)
    Z = jax.lax.dot_general(zacc, ones_f32, (((1,), (0,)), ((), ())),
                            preferred_element_type=jnp.float32)
    e = e3.reshape(_F, _L)

    As = []
    for i in range(_T):
        if i == 0:
            As.append(e)
        else:
            zeros = jnp.zeros((_F, i), jnp.float32)
            bodyp = e[:, :_L - 1 - i]
            tailp = jnp.sum(e[:, _L - 1 - i:], axis=1, keepdims=True)
            As.append(jnp.concatenate([zeros, bodyp, tailp], axis=1))
    A = jnp.stack(As, axis=0).reshape(_T, _N)

    outdot = jax.lax.dot_general(Xm, A, (((1,), (1,)), ((), ())),
                                 preferred_element_type=jnp.float32)
    G = (w / Z[0:1, 0:1]) * outdot + (0.5 - w) * Qt
    out_ref[0] = G.T


def kernel(x, w):
    B = x.shape[0]
    w2 = jnp.asarray(w, jnp.float32).reshape(1, 1)
    out = pl.pallas_call(
        _attn_kernel,
        grid=(B,),
        in_specs=[
            pl.BlockSpec((1, _C, _F, _L), lambda b: (b, 0, 0, 0)),
            pl.BlockSpec((1, 1), lambda b: (0, 0)),
        ],
        out_specs=pl.BlockSpec((1, _T, _C), lambda b: (b, 0, 0)),
        out_shape=jax.ShapeDtypeStruct((B, _T, _C), jnp.float32),
        compiler_params=pltpu.CompilerParams(
            dimension_semantics=("parallel",)),
    )(x, w2)
    return out.reshape(B, _C, 1, _T)
